# baseline (speedup 1.0000x reference)
import functools

import jax
import jax.numpy as jnp
from jax import lax
from jax.experimental import pallas as pl
from jax.experimental.pallas import tpu as pltpu
from jax.experimental.pallas import tpu_sc as plsc

S, D, H, E = 2048, 768, 3072, 8
A = 2 * S
BLK = 1024
NBLK = A // BLK + E
NROWS = NBLK * BLK
TS = 256
CH = 1024
NCH = A // CH
NTILES = 32
TPW = S // NTILES


def _gate_route_kernel(x_ref, gnw_ref, gnb_ref, gw_ref, gb_ref,
                       pb0_ref, pb1_ref, pos_ref, be_ref, act_ref,
                       m_scr, r_scr, carry_scr):
    step = pl.program_id(0)

    @pl.when(step == 0)
    def _gating():
        x = x_ref[...]
        m = jnp.mean(x, axis=1, keepdims=True)
        xc = x - m
        v = jnp.mean(xc * xc, axis=1, keepdims=True)
        ln = xc * lax.rsqrt(v + 1e-5) * gnw_ref[...] + gnb_ref[...]
        logits = lax.dot_general(
            ln, gw_ref[...], dimension_numbers=(((1,), (1,)), ((), ())),
            preferred_element_type=jnp.float32,
        ) + gb_ref[...]
        mx = jnp.max(logits, axis=1, keepdims=True)
        ex = jnp.exp(logits - mx)
        p = ex / jnp.sum(ex, axis=1, keepdims=True)
        cols = lax.broadcasted_iota(jnp.int32, (S, E), 1)
        p0 = jnp.max(p, axis=1, keepdims=True)
        i0 = jnp.min(jnp.where(p == p0, cols, E), axis=1, keepdims=True)
        sel0 = cols == i0
        pm = jnp.where(sel0, -jnp.inf, p)
        p1 = jnp.max(pm, axis=1, keepdims=True)
        i1 = jnp.min(jnp.where(pm == p1, cols, E), axis=1, keepdims=True)
        sel1 = cols == i1
        denom = p0 + p1 + 1e-9
        m_scr[0:S, :] = sel0.astype(jnp.float32)
        m_scr[S:A, :] = sel1.astype(jnp.float32)
        pb0_ref[...] = jnp.broadcast_to(p0 / denom, (S, 16))
        pb1_ref[...] = jnp.broadcast_to(p1 / denom, (S, 16))
        carry_scr[...] = jnp.zeros((1, E), jnp.float32)

    @pl.when(step > 0)
    def _route_chunk():
        c = step - 1
        mc = m_scr[pl.ds(c * CH, CH), :]
        row = lax.broadcasted_iota(jnp.int32, (CH, CH), 0)
        col = lax.broadcasted_iota(jnp.int32, (CH, CH), 1)
        tri = (row > col).astype(jnp.bfloat16)
        rc = lax.dot_general(
            tri, mc.astype(jnp.bfloat16),
            dimension_numbers=(((1,), (0,)), ((), ())),
            preferred_element_type=jnp.float32,
        )
        r_scr[pl.ds(c * CH, CH), :] = rc + carry_scr[...]
        carry_scr[...] = carry_scr[...] + jnp.sum(mc, axis=0, keepdims=True)

    @pl.when(step == NCH)
    def _finalize():
        counts = carry_scr[...].astype(jnp.int32)
        cb = (counts + (BLK - 1)) // BLK
        erow = lax.broadcasted_iota(jnp.int32, (E, E), 0)
        ecol = lax.broadcasted_iota(jnp.int32, (E, E), 1)
        m8 = (erow < ecol).astype(jnp.float32)
        boff = lax.dot_general(
            cb.astype(jnp.float32), m8,
            dimension_numbers=(((1,), (0,)), ((), ())),
            preferred_element_type=jnp.float32,
            precision=lax.Precision.HIGHEST,
        )
        roff = boff * float(BLK)
        boff_col = jnp.reshape(boff, (E, 1)).astype(jnp.int32)
        bids = lax.broadcasted_iota(jnp.int32, (E, NBLK), 1)
        be = jnp.sum((boff_col <= bids).astype(jnp.int32), axis=0,
                     keepdims=True) - 1
        be_ref[...] = jnp.clip(be, 0, E - 1)
        total_blocks = jnp.sum(cb)
        act_ref[...] = (lax.broadcasted_iota(jnp.int32, (1, NBLK), 1)
                        < total_blocks).astype(jnp.int32)
        pos = jnp.sum((r_scr[...] + roff) * m_scr[...], axis=1)
        pos_ref[...] = jnp.reshape(pos, (A // 128, 128)).astype(jnp.int32)


HT = 512


def _expert_kernel(be_ref, act_ref, xs_ref, f1w_ref, f1b_ref, f2w_ref,
                   f2b_ref, lnw_ref, lnb_ref, o_ref):
    b = pl.program_id(0)

    @pl.when(act_ref[b] == 1)
    def _():
        x = xs_ref[...]
        xb = x.astype(jnp.bfloat16)
        o = jnp.zeros((BLK, D), jnp.float32)
        for j in range(H // HT):
            hs = pl.ds(j * HT, HT)
            w1j = f1w_ref[0, hs, :].astype(jnp.bfloat16)
            hj = lax.dot_general(
                xb, w1j, dimension_numbers=(((1,), (1,)), ((), ())),
                preferred_element_type=jnp.float32,
            ) + f1b_ref[0, 0, hs]
            hj = 0.5 * hj * (1.0 + lax.erf(hj * 0.7071067811865476))
            w2j = f2w_ref[0, :, hs].astype(jnp.bfloat16)
            o = o + lax.dot_general(
                hj.astype(jnp.bfloat16), w2j,
                dimension_numbers=(((1,), (1,)), ((), ())),
                preferred_element_type=jnp.float32,
            )
        o = o + f2b_ref[0] + x
        m = jnp.mean(o, axis=1, keepdims=True)
        oc = o - m
        v = jnp.mean(oc * oc, axis=1, keepdims=True)
        o_ref[...] = oc * lax.rsqrt(v + 1e-5) * lnw_ref[0] + lnb_ref[0]


@functools.lru_cache(maxsize=None)
def _make_sc_mesh():
    return plsc.VectorSubcoreMesh(core_axis_name="c", subcore_axis_name="s",
                                  num_cores=2, num_subcores=16)


@functools.lru_cache(maxsize=None)
def _make_scatter_x():
    @functools.partial(
        pl.kernel,
        out_type=jax.ShapeDtypeStruct((NROWS, D), jnp.float32),
        mesh=_make_sc_mesh(),
        scratch_types=[
            pltpu.VMEM((TPW,), jnp.int32),
            pltpu.VMEM((TPW,), jnp.int32),
            pltpu.VMEM((TPW, D), jnp.float32),
            pltpu.SemaphoreType.DMA,
            pltpu.SemaphoreType.DMA,
            pltpu.SemaphoreType.DMA,
        ],
    )
    def _scatter_x(x_hbm, pos_hbm, xs_hbm, idx0_v, idx1_v, rows_v,
                   sem_in, s0, s1):
        wid = lax.axis_index("s") * 2 + lax.axis_index("c")
        base = wid * TPW
        cx = pltpu.async_copy(x_hbm.at[pl.ds(base, TPW), :], rows_v, sem_in)
        pltpu.sync_copy(pos_hbm.at[pl.ds(base, TPW)], idx0_v)
        pltpu.sync_copy(pos_hbm.at[pl.ds(S + base, TPW)], idx1_v)
        cx.wait()
        c0 = pltpu.async_copy(rows_v, xs_hbm.at[idx0_v], s0)
        c1 = pltpu.async_copy(rows_v, xs_hbm.at[idx1_v], s1)
        c0.wait()
        c1.wait()

    return _scatter_x


@functools.lru_cache(maxsize=None)
def _make_gather_combine():
    @functools.partial(
        pl.kernel,
        out_type=jax.ShapeDtypeStruct((S, D), jnp.float32),
        mesh=_make_sc_mesh(),
        scratch_types=[
            pltpu.VMEM((TPW,), jnp.int32),
            pltpu.VMEM((TPW,), jnp.int32),
            pltpu.VMEM((TPW, D), jnp.float32),
            pltpu.VMEM((TPW, D), jnp.float32),
            pltpu.VMEM((TPW, 16), jnp.float32),
            pltpu.VMEM((TPW, 16), jnp.float32),
            pltpu.SemaphoreType.DMA,
            pltpu.SemaphoreType.DMA,
        ],
    )
    def _gather_combine(rows_hbm, pos_hbm, pb0_hbm, pb1_hbm, out_hbm,
                        i0_v, i1_v, b0_v, b1_v, p0_v, p1_v, s0, s1):
        wid = lax.axis_index("s") * 2 + lax.axis_index("c")
        base = wid * TPW
        pltpu.sync_copy(pos_hbm.at[pl.ds(base, TPW)], i0_v)
        pltpu.sync_copy(pos_hbm.at[pl.ds(S + base, TPW)], i1_v)
        c0 = pltpu.async_copy(rows_hbm.at[i0_v], b0_v, s0)
        c1 = pltpu.async_copy(rows_hbm.at[i1_v], b1_v, s1)
        pltpu.sync_copy(pb0_hbm.at[pl.ds(base, TPW), :], p0_v)
        pltpu.sync_copy(pb1_hbm.at[pl.ds(base, TPW), :], p1_v)
        c0.wait()
        c1.wait()

        def body(i, carry):
            p0 = p0_v[i, :]
            p1 = p1_v[i, :]
            for j in range(D // 16):
                sl = pl.ds(j * 16, 16)
                b0_v[i, sl] = b0_v[i, sl] * p0 + b1_v[i, sl] * p1
            return carry

        lax.fori_loop(0, TPW, body, 0)
        pltpu.sync_copy(b0_v, out_hbm.at[pl.ds(base, TPW), :])

    return _gather_combine


@jax.jit
def kernel(x, fc1_w, fc1_b, fc2_w, fc2_b, ln_w, ln_b, gn_w, gn_b, gate_w, gate_b):
    orig_shape = x.shape
    x2 = x.reshape(S, D)

    pb0, pb1, pos32, be2, act2 = pl.pallas_call(
        _gate_route_kernel,
        grid=(NCH + 1,),
        in_specs=[
            pl.BlockSpec((S, D), lambda c: (0, 0)),
            pl.BlockSpec((1, D), lambda c: (0, 0)),
            pl.BlockSpec((1, D), lambda c: (0, 0)),
            pl.BlockSpec((E, D), lambda c: (0, 0)),
            pl.BlockSpec((1, E), lambda c: (0, 0)),
        ],
        out_specs=(pl.BlockSpec((S, 16), lambda c: (0, 0)),
                   pl.BlockSpec((S, 16), lambda c: (0, 0)),
                   pl.BlockSpec((A // 128, 128), lambda c: (0, 0)),
                   pl.BlockSpec((1, NBLK), lambda c: (0, 0)),
                   pl.BlockSpec((1, NBLK), lambda c: (0, 0))),
        out_shape=(jax.ShapeDtypeStruct((S, 16), jnp.float32),
                   jax.ShapeDtypeStruct((S, 16), jnp.float32),
                   jax.ShapeDtypeStruct((A // 128, 128), jnp.int32),
                   jax.ShapeDtypeStruct((1, NBLK), jnp.int32),
                   jax.ShapeDtypeStruct((1, NBLK), jnp.int32)),
        scratch_shapes=[pltpu.VMEM((A, E), jnp.float32),
                        pltpu.VMEM((A, E), jnp.float32),
                        pltpu.VMEM((1, E), jnp.float32)],
        compiler_params=pltpu.CompilerParams(
            dimension_semantics=("arbitrary",)),
    )(x2, gn_w.reshape(1, D), gn_b.reshape(1, D), gate_w,
      gate_b.reshape(1, E))
    pos_flat = pos32.reshape(A)
    be_flat = be2.reshape(NBLK)
    act_flat = act2.reshape(NBLK)

    xs = _make_scatter_x()(x2, pos_flat)

    rows = pl.pallas_call(
        _expert_kernel,
        grid_spec=pltpu.PrefetchScalarGridSpec(
            num_scalar_prefetch=2,
            grid=(NBLK,),
            in_specs=[
                pl.BlockSpec((BLK, D), lambda b, be, act: (b, 0)),
                pl.BlockSpec((1, H, D), lambda b, be, act: (be[b], 0, 0)),
                pl.BlockSpec((1, 1, H), lambda b, be, act: (be[b], 0, 0)),
                pl.BlockSpec((1, D, H), lambda b, be, act: (be[b], 0, 0)),
                pl.BlockSpec((1, 1, D), lambda b, be, act: (be[b], 0, 0)),
                pl.BlockSpec((1, 1, D), lambda b, be, act: (be[b], 0, 0)),
                pl.BlockSpec((1, 1, D), lambda b, be, act: (be[b], 0, 0)),
            ],
            out_specs=pl.BlockSpec((BLK, D), lambda b, be, act: (b, 0)),
        ),
        out_shape=jax.ShapeDtypeStruct((NROWS, D), jnp.float32),
        compiler_params=pltpu.CompilerParams(
            dimension_semantics=("arbitrary",)),
    )(be_flat, act_flat, xs, fc1_w, fc1_b.reshape(E, 1, H), fc2_w,
      fc2_b.reshape(E, 1, D), ln_w.reshape(E, 1, D), ln_b.reshape(E, 1, D))

    out = _make_gather_combine()(rows, pos_flat, pb0, pb1)
    return out.reshape(orig_shape)


# speedup vs baseline: 1.1317x; 1.1317x over previous
import functools

import jax
import jax.numpy as jnp
from jax import lax
from jax.experimental import pallas as pl
from jax.experimental.pallas import tpu as pltpu
from jax.experimental.pallas import tpu_sc as plsc

S, D, H, E = 2048, 768, 3072, 8
A = 2 * S
BLK = 512
NBLK = A // BLK + E
NROWS = NBLK * BLK
TS = 256
CH = 1024
NCH = A // CH
NTILES = 32
TPW = S // NTILES


def _gate_route_kernel(x_ref, gnw_ref, gnb_ref, gw_ref, gb_ref,
                       pb0_ref, pb1_ref, pos_ref, be_ref, act_ref,
                       m_scr, r_scr, carry_scr):
    step = pl.program_id(0)

    @pl.when(step == 0)
    def _gating():
        x = x_ref[...]
        m = jnp.mean(x, axis=1, keepdims=True)
        xc = x - m
        v = jnp.mean(xc * xc, axis=1, keepdims=True)
        ln = xc * lax.rsqrt(v + 1e-5) * gnw_ref[...] + gnb_ref[...]
        logits = lax.dot_general(
            ln, gw_ref[...], dimension_numbers=(((1,), (1,)), ((), ())),
            preferred_element_type=jnp.float32,
        ) + gb_ref[...]
        mx = jnp.max(logits, axis=1, keepdims=True)
        ex = jnp.exp(logits - mx)
        p = ex / jnp.sum(ex, axis=1, keepdims=True)
        cols = lax.broadcasted_iota(jnp.int32, (S, E), 1)
        p0 = jnp.max(p, axis=1, keepdims=True)
        i0 = jnp.min(jnp.where(p == p0, cols, E), axis=1, keepdims=True)
        sel0 = cols == i0
        pm = jnp.where(sel0, -jnp.inf, p)
        p1 = jnp.max(pm, axis=1, keepdims=True)
        i1 = jnp.min(jnp.where(pm == p1, cols, E), axis=1, keepdims=True)
        sel1 = cols == i1
        denom = p0 + p1 + 1e-9
        m_scr[0:S, :] = sel0.astype(jnp.float32)
        m_scr[S:A, :] = sel1.astype(jnp.float32)
        pb0_ref[...] = jnp.broadcast_to(p0 / denom, (S, 16))
        pb1_ref[...] = jnp.broadcast_to(p1 / denom, (S, 16))
        carry_scr[...] = jnp.zeros((1, E), jnp.float32)

    @pl.when(step > 0)
    def _route_chunk():
        c = step - 1
        mc = m_scr[pl.ds(c * CH, CH), :]
        row = lax.broadcasted_iota(jnp.int32, (CH, CH), 0)
        col = lax.broadcasted_iota(jnp.int32, (CH, CH), 1)
        tri = (row > col).astype(jnp.bfloat16)
        rc = lax.dot_general(
            tri, mc.astype(jnp.bfloat16),
            dimension_numbers=(((1,), (0,)), ((), ())),
            preferred_element_type=jnp.float32,
        )
        r_scr[pl.ds(c * CH, CH), :] = rc + carry_scr[...]
        carry_scr[...] = carry_scr[...] + jnp.sum(mc, axis=0, keepdims=True)

    @pl.when(step == NCH)
    def _finalize():
        counts = carry_scr[...].astype(jnp.int32)
        cb = (counts + (BLK - 1)) // BLK
        erow = lax.broadcasted_iota(jnp.int32, (E, E), 0)
        ecol = lax.broadcasted_iota(jnp.int32, (E, E), 1)
        m8 = (erow < ecol).astype(jnp.float32)
        boff = lax.dot_general(
            cb.astype(jnp.float32), m8,
            dimension_numbers=(((1,), (0,)), ((), ())),
            preferred_element_type=jnp.float32,
            precision=lax.Precision.HIGHEST,
        )
        roff = boff * float(BLK)
        boff_col = jnp.reshape(boff, (E, 1)).astype(jnp.int32)
        bids = lax.broadcasted_iota(jnp.int32, (E, NBLK), 1)
        be = jnp.sum((boff_col <= bids).astype(jnp.int32), axis=0,
                     keepdims=True) - 1
        be_ref[...] = jnp.clip(be, 0, E - 1)
        total_blocks = jnp.sum(cb)
        act_ref[...] = (lax.broadcasted_iota(jnp.int32, (1, NBLK), 1)
                        < total_blocks).astype(jnp.int32)
        pos = jnp.sum((r_scr[...] + roff) * m_scr[...], axis=1)
        pos_ref[...] = jnp.reshape(pos, (A // 128, 128)).astype(jnp.int32)


HT = 1024


def _expert_kernel(be_ref, act_ref, xs_ref, f1w_ref, f1b_ref, f2w_ref,
                   f2b_ref, lnw_ref, lnb_ref, o_ref):
    b = pl.program_id(0)

    @pl.when(act_ref[b] == 1)
    def _():
        x = xs_ref[...]
        xb = x.astype(jnp.bfloat16)
        o = jnp.zeros((BLK, D), jnp.float32)
        for j in range(H // HT):
            hs = pl.ds(j * HT, HT)
            w1j = f1w_ref[0, hs, :].astype(jnp.bfloat16)
            hj = lax.dot_general(
                xb, w1j, dimension_numbers=(((1,), (1,)), ((), ())),
                preferred_element_type=jnp.float32,
            ) + f1b_ref[0, 0, hs]
            hj = 0.5 * hj * (1.0 + lax.erf(hj * 0.7071067811865476))
            w2j = f2w_ref[0, :, hs].astype(jnp.bfloat16)
            o = o + lax.dot_general(
                hj.astype(jnp.bfloat16), w2j,
                dimension_numbers=(((1,), (1,)), ((), ())),
                preferred_element_type=jnp.float32,
            )
        o = o + f2b_ref[0] + x
        m = jnp.mean(o, axis=1, keepdims=True)
        oc = o - m
        v = jnp.mean(oc * oc, axis=1, keepdims=True)
        o_ref[...] = oc * lax.rsqrt(v + 1e-5) * lnw_ref[0] + lnb_ref[0]


@functools.lru_cache(maxsize=None)
def _make_sc_mesh():
    return plsc.VectorSubcoreMesh(core_axis_name="c", subcore_axis_name="s",
                                  num_cores=2, num_subcores=16)


@functools.lru_cache(maxsize=None)
def _make_scatter_x():
    @functools.partial(
        pl.kernel,
        out_type=jax.ShapeDtypeStruct((NROWS, D), jnp.float32),
        mesh=_make_sc_mesh(),
        scratch_types=[
            pltpu.VMEM((TPW,), jnp.int32),
            pltpu.VMEM((TPW,), jnp.int32),
            pltpu.VMEM((TPW, D), jnp.float32),
            pltpu.SemaphoreType.DMA,
            pltpu.SemaphoreType.DMA,
            pltpu.SemaphoreType.DMA,
        ],
    )
    def _scatter_x(x_hbm, pos_hbm, xs_hbm, idx0_v, idx1_v, rows_v,
                   sem_in, s0, s1):
        wid = lax.axis_index("s") * 2 + lax.axis_index("c")
        base = wid * TPW
        cx = pltpu.async_copy(x_hbm.at[pl.ds(base, TPW), :], rows_v, sem_in)
        pltpu.sync_copy(pos_hbm.at[pl.ds(base, TPW)], idx0_v)
        pltpu.sync_copy(pos_hbm.at[pl.ds(S + base, TPW)], idx1_v)
        cx.wait()
        c0 = pltpu.async_copy(rows_v, xs_hbm.at[idx0_v], s0)
        c1 = pltpu.async_copy(rows_v, xs_hbm.at[idx1_v], s1)
        c0.wait()
        c1.wait()

    return _scatter_x


@functools.lru_cache(maxsize=None)
def _make_gather_combine():
    @functools.partial(
        pl.kernel,
        out_type=jax.ShapeDtypeStruct((S, D), jnp.float32),
        mesh=_make_sc_mesh(),
        scratch_types=[
            pltpu.VMEM((TPW,), jnp.int32),
            pltpu.VMEM((TPW,), jnp.int32),
            pltpu.VMEM((TPW, D), jnp.float32),
            pltpu.VMEM((TPW, D), jnp.float32),
            pltpu.VMEM((TPW, 16), jnp.float32),
            pltpu.VMEM((TPW, 16), jnp.float32),
            pltpu.SemaphoreType.DMA,
            pltpu.SemaphoreType.DMA,
        ],
    )
    def _gather_combine(rows_hbm, pos_hbm, pb0_hbm, pb1_hbm, out_hbm,
                        i0_v, i1_v, b0_v, b1_v, p0_v, p1_v, s0, s1):
        wid = lax.axis_index("s") * 2 + lax.axis_index("c")
        base = wid * TPW
        pltpu.sync_copy(pos_hbm.at[pl.ds(base, TPW)], i0_v)
        pltpu.sync_copy(pos_hbm.at[pl.ds(S + base, TPW)], i1_v)
        c0 = pltpu.async_copy(rows_hbm.at[i0_v], b0_v, s0)
        c1 = pltpu.async_copy(rows_hbm.at[i1_v], b1_v, s1)
        pltpu.sync_copy(pb0_hbm.at[pl.ds(base, TPW), :], p0_v)
        pltpu.sync_copy(pb1_hbm.at[pl.ds(base, TPW), :], p1_v)
        c0.wait()
        c1.wait()

        def body(i, carry):
            p0 = p0_v[i, :]
            p1 = p1_v[i, :]
            for j in range(D // 16):
                sl = pl.ds(j * 16, 16)
                b0_v[i, sl] = b0_v[i, sl] * p0 + b1_v[i, sl] * p1
            return carry

        lax.fori_loop(0, TPW, body, 0)
        pltpu.sync_copy(b0_v, out_hbm.at[pl.ds(base, TPW), :])

    return _gather_combine


@jax.jit
def kernel(x, fc1_w, fc1_b, fc2_w, fc2_b, ln_w, ln_b, gn_w, gn_b, gate_w, gate_b):
    orig_shape = x.shape
    x2 = x.reshape(S, D)

    pb0, pb1, pos32, be2, act2 = pl.pallas_call(
        _gate_route_kernel,
        grid=(NCH + 1,),
        in_specs=[
            pl.BlockSpec((S, D), lambda c: (0, 0)),
            pl.BlockSpec((1, D), lambda c: (0, 0)),
            pl.BlockSpec((1, D), lambda c: (0, 0)),
            pl.BlockSpec((E, D), lambda c: (0, 0)),
            pl.BlockSpec((1, E), lambda c: (0, 0)),
        ],
        out_specs=(pl.BlockSpec((S, 16), lambda c: (0, 0)),
                   pl.BlockSpec((S, 16), lambda c: (0, 0)),
                   pl.BlockSpec((A // 128, 128), lambda c: (0, 0)),
                   pl.BlockSpec((1, NBLK), lambda c: (0, 0)),
                   pl.BlockSpec((1, NBLK), lambda c: (0, 0))),
        out_shape=(jax.ShapeDtypeStruct((S, 16), jnp.float32),
                   jax.ShapeDtypeStruct((S, 16), jnp.float32),
                   jax.ShapeDtypeStruct((A // 128, 128), jnp.int32),
                   jax.ShapeDtypeStruct((1, NBLK), jnp.int32),
                   jax.ShapeDtypeStruct((1, NBLK), jnp.int32)),
        scratch_shapes=[pltpu.VMEM((A, E), jnp.float32),
                        pltpu.VMEM((A, E), jnp.float32),
                        pltpu.VMEM((1, E), jnp.float32)],
        compiler_params=pltpu.CompilerParams(
            dimension_semantics=("arbitrary",)),
    )(x2, gn_w.reshape(1, D), gn_b.reshape(1, D), gate_w,
      gate_b.reshape(1, E))
    pos_flat = pos32.reshape(A)
    be_flat = be2.reshape(NBLK)
    act_flat = act2.reshape(NBLK)

    xs = _make_scatter_x()(x2, pos_flat)

    rows = pl.pallas_call(
        _expert_kernel,
        grid_spec=pltpu.PrefetchScalarGridSpec(
            num_scalar_prefetch=2,
            grid=(NBLK,),
            in_specs=[
                pl.BlockSpec((BLK, D), lambda b, be, act: (b, 0)),
                pl.BlockSpec((1, H, D), lambda b, be, act: (be[b], 0, 0)),
                pl.BlockSpec((1, 1, H), lambda b, be, act: (be[b], 0, 0)),
                pl.BlockSpec((1, D, H), lambda b, be, act: (be[b], 0, 0)),
                pl.BlockSpec((1, 1, D), lambda b, be, act: (be[b], 0, 0)),
                pl.BlockSpec((1, 1, D), lambda b, be, act: (be[b], 0, 0)),
                pl.BlockSpec((1, 1, D), lambda b, be, act: (be[b], 0, 0)),
            ],
            out_specs=pl.BlockSpec((BLK, D), lambda b, be, act: (b, 0)),
        ),
        out_shape=jax.ShapeDtypeStruct((NROWS, D), jnp.float32),
        compiler_params=pltpu.CompilerParams(
            dimension_semantics=("arbitrary",)),
    )(be_flat, act_flat, xs, fc1_w, fc1_b.reshape(E, 1, H), fc2_w,
      fc2_b.reshape(E, 1, D), ln_w.reshape(E, 1, D), ln_b.reshape(E, 1, D))

    out = _make_gather_combine()(rows, pos_flat, pb0, pb1)
    return out.reshape(orig_shape)
